# baseline (speedup 1.0000x reference)
import functools
import math

import jax
import jax.numpy as jnp
from jax import lax
from jax.experimental import pallas as pl
from jax.experimental.pallas import tpu as pltpu
from jax.experimental.pallas import tpu_sc as plsc

D_MODEL = 64
SCALE = math.sqrt(D_MODEL)

NC, NS, L = 2, 16, 16
NW = NC * NS

IDXW = 128
NBUF = 4
ROW_UNROLL = 8


def _make_kernel(B0, S, V):
    seq_per_w = B0 // NW
    G = seq_per_w
    assert G % NBUF == 0
    sub = [(0, IDXW), (IDXW, S - IDXW)]
    mesh = plsc.VectorSubcoreMesh(core_axis_name="c", subcore_axis_name="s")

    scratch = [pltpu.VMEM((seq_per_w, S), jnp.int32)]
    scratch += [pltpu.VMEM((S, D_MODEL), jnp.float32) for _ in range(NBUF)]
    scratch += [pltpu.SemaphoreType.DMA for _ in range(2 * NBUF)]

    @functools.partial(
        pl.kernel,
        out_type=jax.ShapeDtypeStruct((B0, S, D_MODEL), jnp.float32),
        mesh=mesh,
        scratch_types=scratch,
        compiler_params=pltpu.CompilerParams(use_tc_tiling_on_sc=False),
    )
    def emb_kernel(x_hbm, w_hbm, out_hbm, idx_v, *bufs_and_sems):
        rows = bufs_and_sems[:NBUF]
        gsem = bufs_and_sems[NBUF:2 * NBUF]
        ssem = bufs_and_sems[2 * NBUF:]

        wid = lax.axis_index("s") * NC + lax.axis_index("c")
        s0 = wid * seq_per_w

        pltpu.sync_copy(x_hbm.at[pl.ds(s0, seq_per_w)], idx_v)

        def fire_gather(gg, b):
            for off, cnt in sub:
                pltpu.async_copy(
                    w_hbm.at[idx_v.at[gg, pl.ds(off, cnt)]],
                    rows[b].at[pl.ds(off, cnt)],
                    gsem[b],
                )

        def drain_gather(b):
            for off, cnt in sub:
                pltpu.make_async_copy(
                    w_hbm.at[idx_v.at[0, pl.ds(off, cnt)]],
                    rows[b].at[pl.ds(off, cnt)],
                    gsem[b],
                ).wait()

        def drain_scatter(b):
            pltpu.make_async_copy(rows[b], out_hbm.at[0], ssem[b]).wait()

        fire_gather(0, 0)

        def outer(g0, carry):
            for b in range(NBUF):
                gg = g0 * NBUF + b
                nb = (b + 1) % NBUF

                @pl.when(gg >= NBUF - 1)
                def _():
                    drain_scatter(nb)

                @pl.when(gg + 1 < G)
                def _():
                    fire_gather(gg + 1, nb)

                drain_gather(b)

                buf = rows[b]

                @plsc.parallel_loop(0, S, unroll=ROW_UNROLL)
                def _(r):
                    for k in range(D_MODEL // L):
                        buf[r, pl.ds(k * L, L)] = buf[r, pl.ds(k * L, L)] * SCALE

                pltpu.async_copy(buf, out_hbm.at[s0 + gg], ssem[b])
            return carry

        lax.fori_loop(0, G // NBUF, outer, 0)

        for b in range(1, NBUF):
            drain_scatter(b)

    return emb_kernel


def kernel(x, W):
    B0, S = x.shape
    V = W.shape[0]
    return _make_kernel(B0, S, V)(x.astype(jnp.int32), W)


# speedup vs baseline: 1.2310x; 1.2310x over previous
import functools
import math

import jax
import jax.numpy as jnp
from jax import lax
from jax.experimental import pallas as pl
from jax.experimental.pallas import tpu as pltpu
from jax.experimental.pallas import tpu_sc as plsc

D_MODEL = 64
DPAD = 128
SCALE = math.sqrt(D_MODEL)

NC, NS, L = 2, 16, 16
NW = NC * NS

NBUF = 4
ROW_UNROLL = 8
SUB = ((0, 104), (104, 96))


def _make_kernel(B0, S, V):
    seq_per_w = B0 // NW
    G = seq_per_w * len(SUB)
    assert G % NBUF == 0
    cmax = max(c for _, c in SUB)
    mesh = plsc.VectorSubcoreMesh(core_axis_name="c", subcore_axis_name="s")

    scratch = [pltpu.VMEM((seq_per_w, S), jnp.int32)]
    scratch += [pltpu.VMEM((cmax, DPAD), jnp.float32) for _ in range(NBUF)]
    scratch += [pltpu.SemaphoreType.DMA for _ in range(2 * NBUF)]

    @functools.partial(
        pl.kernel,
        out_type=jax.ShapeDtypeStruct((B0, S, DPAD), jnp.float32),
        mesh=mesh,
        scratch_types=scratch,
        compiler_params=pltpu.CompilerParams(use_tc_tiling_on_sc=False),
    )
    def emb_kernel(x_hbm, w_hbm, out_hbm, idx_v, *bufs_and_sems):
        rows = bufs_and_sems[:NBUF]
        gsem = bufs_and_sems[NBUF:2 * NBUF]
        ssem = bufs_and_sems[2 * NBUF:]

        wid = lax.axis_index("s") * NC + lax.axis_index("c")
        s0 = wid * seq_per_w

        pltpu.sync_copy(x_hbm.at[pl.ds(s0, seq_per_w)], idx_v)

        def fire_gather(seq, h, b):
            off, cnt = SUB[h]
            pltpu.async_copy(
                w_hbm.at[idx_v.at[seq, pl.ds(off, cnt)]],
                rows[b].at[pl.ds(0, cnt)],
                gsem[b],
            )

        def drain_gather(h, b):
            off, cnt = SUB[h]
            pltpu.make_async_copy(
                w_hbm.at[idx_v.at[0, pl.ds(off, cnt)]],
                rows[b].at[pl.ds(0, cnt)],
                gsem[b],
            ).wait()

        def drain_scatter(h, b):
            off, cnt = SUB[h]
            pltpu.make_async_copy(
                rows[b].at[pl.ds(0, cnt), pl.ds(0, D_MODEL)],
                out_hbm.at[0, pl.ds(off, cnt), pl.ds(0, D_MODEL)],
                ssem[b],
            ).wait()

        fire_gather(0, 0, 0)

        def outer(g0, carry):
            for b in range(NBUF):
                gg = g0 * NBUF + b
                h = b % 2
                nh = (b + 1) % 2
                nb = (b + 1) % NBUF
                lseq = g0 * (NBUF // 2) + b // 2
                seq = s0 + lseq
                nlseq = lseq + (1 if h == 1 else 0)

                @pl.when(gg >= NBUF - 1)
                def _():
                    drain_scatter(nh, nb)

                @pl.when(gg + 1 < G)
                def _():
                    fire_gather(nlseq, nh, nb)

                drain_gather(h, b)

                buf = rows[b]
                off, cnt = SUB[h]

                @plsc.parallel_loop(0, cnt, unroll=ROW_UNROLL)
                def _(r):
                    for k in range(D_MODEL // L):
                        buf[r, pl.ds(k * L, L)] = buf[r, pl.ds(k * L, L)] * SCALE

                pltpu.async_copy(
                    buf.at[pl.ds(0, cnt), pl.ds(0, D_MODEL)],
                    out_hbm.at[seq, pl.ds(off, cnt), pl.ds(0, D_MODEL)],
                    ssem[b],
                )
            return carry

        lax.fori_loop(0, G // NBUF, outer, 0)

        for b in range(1, NBUF):
            drain_scatter(b % 2, b)

    return emb_kernel


def kernel(x, W):
    B0, S = x.shape
    V = W.shape[0]
    Wp = jnp.pad(W, ((0, 0), (0, DPAD - D_MODEL)))
    out = _make_kernel(B0, S, V)(x.astype(jnp.int32), Wp)
    return out[:, :, :D_MODEL]
